# baseline (speedup 1.0000x reference)
import functools

import jax
import jax.numpy as jnp
from jax.experimental import pallas as pl
from jax.experimental.pallas import tpu as pltpu

_K = 2
_GC = 0.7978845608028654


@functools.partial(jax.jit, static_argnames=())
def kernel(x, W_router, W_up, W_down, b_up, b_down):
    b, s, d = x.shape
    e, u, _ = W_up.shape
    k = _K
    bs = b * s
    x2 = x.reshape(bs, d)

    logits = x2 @ W_router
    top_logits, indices = jax.lax.top_k(logits, k)
    rw = jax.nn.softmax(top_logits, axis=-1)
    flat_e = indices.reshape(-1).astype(jnp.int32)
    flat_t = (jnp.arange(bs * k, dtype=jnp.int32) // k)
    flat_w = rw.reshape(-1)
    e_s = jnp.sort(flat_e)
    rw3 = jnp.zeros((e, 1, 128), jnp.float32).at[flat_e, 0, flat_t].add(flat_w)
    xT = x2.T

    npairs = bs * k

    grid_spec = pltpu.PrefetchScalarGridSpec(
        num_scalar_prefetch=1,
        grid=(npairs,),
        in_specs=[
            pl.BlockSpec((d, bs), lambda j, er: (0, 0)),
            pl.BlockSpec((1, u, d), lambda j, er: (er[j], 0, 0)),
            pl.BlockSpec((1, d, u), lambda j, er: (er[j], 0, 0)),
            pl.BlockSpec((1, u, 1), lambda j, er: (er[j], 0, 0)),
            pl.BlockSpec((1, d, 1), lambda j, er: (er[j], 0, 0)),
            pl.BlockSpec((1, 1, 128), lambda j, er: (er[j], 0, 0)),
        ],
        out_specs=pl.BlockSpec((d, bs), lambda j, er: (0, 0)),
    )

    def ffn_body(e_ref, xt_ref, wu_ref, wd_ref, bu_ref, bd_ref, rw_ref,
                 out_ref):
        j = pl.program_id(0)
        prev = e_ref[jnp.maximum(j - 1, 0)]
        first = (j == 0) | (e_ref[j] != prev)

        @pl.when(first)
        def _process_expert():
            ht = jax.lax.dot_general(wu_ref[0], xt_ref[...],
                                     (((1,), (0,)), ((), ())),
                                     preferred_element_type=jnp.float32)
            ht = ht + bu_ref[0]
            inner = _GC * (ht + 0.044715 * (ht * ht * ht))
            ht = ht * (1.0 / (1.0 + jnp.exp(-2.0 * inner)))
            ot = jax.lax.dot_general(wd_ref[0], ht,
                                     (((1,), (0,)), ((), ())),
                                     preferred_element_type=jnp.float32)
            ot = (ot + bd_ref[0]) * rw_ref[0, :, 0:bs]

            @pl.when(j == 0)
            def _init():
                out_ref[...] = ot

            @pl.when(j > 0)
            def _acc():
                out_ref[...] = out_ref[...] + ot

    outT = pl.pallas_call(
        ffn_body,
        grid_spec=grid_spec,
        out_shape=jax.ShapeDtypeStruct((d, bs), jnp.float32),
        compiler_params=pltpu.CompilerParams(
            dimension_semantics=("arbitrary",),
        ),
    )(e_s, xT, W_up, W_down,
      b_up.reshape(e, u, 1), b_down.reshape(e, d, 1), rw3)
    return outT.T.reshape(b, s, d)


# speedup vs baseline: 1.3085x; 1.3085x over previous
import functools

import jax
import jax.numpy as jnp
from jax.experimental import pallas as pl
from jax.experimental.pallas import tpu as pltpu

_K = 2
_GC = 0.7978845608028654


@functools.partial(jax.jit, static_argnames=())
def kernel(x, W_router, W_up, W_down, b_up, b_down):
    b, s, d = x.shape
    e, u, _ = W_up.shape
    k = _K
    bs = b * s
    x2 = x.reshape(bs, d)

    logits = x2 @ W_router
    top_logits, indices = jax.lax.top_k(logits, k)
    rw = jax.nn.softmax(top_logits, axis=-1)
    flat_e = indices.reshape(-1).astype(jnp.int32)
    flat_t = (jnp.arange(bs * k, dtype=jnp.int32) // k)
    flat_w = rw.reshape(-1)
    e_s = jnp.sort(flat_e)
    rw3 = jnp.zeros((e, bs, 128), jnp.float32).at[flat_e, flat_t, 0].add(flat_w)

    npairs = bs * k

    grid_spec = pltpu.PrefetchScalarGridSpec(
        num_scalar_prefetch=1,
        grid=(npairs,),
        in_specs=[
            pl.BlockSpec((bs, d), lambda j, er: (0, 0)),
            pl.BlockSpec((1, u, d), lambda j, er: (er[j], 0, 0)),
            pl.BlockSpec((1, d, u), lambda j, er: (er[j], 0, 0)),
            pl.BlockSpec((1, 1, u), lambda j, er: (er[j], 0, 0)),
            pl.BlockSpec((1, 1, d), lambda j, er: (er[j], 0, 0)),
            pl.BlockSpec((1, bs, 128), lambda j, er: (er[j], 0, 0)),
        ],
        out_specs=pl.BlockSpec((bs, d), lambda j, er: (0, 0)),
    )

    def ffn_body(e_ref, x_ref, wu_ref, wd_ref, bu_ref, bd_ref, rw_ref,
                 out_ref):
        j = pl.program_id(0)
        prev = e_ref[jnp.maximum(j - 1, 0)]
        first = (j == 0) | (e_ref[j] != prev)

        @pl.when(first)
        def _process_expert():
            h = jax.lax.dot_general(x_ref[...], wu_ref[0],
                                    (((1,), (1,)), ((), ())),
                                    precision=jax.lax.Precision.DEFAULT,
                                    preferred_element_type=jnp.float32)
            h = h + bu_ref[0]
            inner = _GC * (h + 0.044715 * (h * h * h))
            h = h * (1.0 / (1.0 + jnp.exp(-2.0 * inner)))
            o = jax.lax.dot_general(h, wd_ref[0],
                                    (((1,), (1,)), ((), ())),
                                    precision=jax.lax.Precision.DEFAULT,
                                    preferred_element_type=jnp.float32)
            o = (o + bd_ref[0]) * rw_ref[0, :, 0:1]

            @pl.when(j == 0)
            def _init():
                out_ref[...] = o

            @pl.when(j > 0)
            def _acc():
                out_ref[...] = out_ref[...] + o

    out = pl.pallas_call(
        ffn_body,
        grid_spec=grid_spec,
        out_shape=jax.ShapeDtypeStruct((bs, d), jnp.float32),
        compiler_params=pltpu.CompilerParams(
            dimension_semantics=("arbitrary",),
        ),
    )(e_s, x2, W_up, W_down,
      b_up.reshape(e, 1, u), b_down.reshape(e, 1, d), rw3)
    return out.reshape(b, s, d)


# speedup vs baseline: 1.3960x; 1.0668x over previous
import functools

import jax
import jax.numpy as jnp
from jax.experimental import pallas as pl
from jax.experimental.pallas import tpu as pltpu

_E = 64
_K = 2
_UT = 2048


def _ffn_body(e_ref, t_ref, x_ref, wu_ref, wd_ref, bu_ref, bd_ref, w_ref,
              out_ref):
    i = pl.program_id(0)
    j = pl.program_id(1)

    @pl.when((i == 0) & (j == 0))
    def _init():
        out_ref[...] = jnp.zeros_like(out_ref)

    t = t_ref[j]
    w = w_ref[j, 0]
    xt = x_ref[pl.ds(t, 1), :]
    h = jax.lax.dot_general(xt, wu_ref[0], (((1,), (1,)), ((), ())),
                            precision=jax.lax.Precision.DEFAULT,
                            preferred_element_type=jnp.float32)
    h = h + bu_ref[0]
    inner = 0.7978845608028654 * (h + 0.044715 * (h * h * h))
    h = h * (1.0 / (1.0 + jnp.exp(-2.0 * inner)))
    o = jax.lax.dot_general(h, wd_ref[0], (((1,), (1,)), ((), ())),
                            precision=jax.lax.Precision.DEFAULT,
                            preferred_element_type=jnp.float32)
    o = o + jnp.where(i == 0, 1.0, 0.0) * bd_ref[0]
    out_ref[pl.ds(t, 1), :] = out_ref[pl.ds(t, 1), :] + w * o


@functools.partial(jax.jit, static_argnames=())
def kernel(x, W_router, W_up, W_down, b_up, b_down):
    b, s, d = x.shape
    e, u, _ = W_up.shape
    k = _K
    x2 = x.reshape(b * s, d)

    logits = x2 @ W_router
    top_logits, indices = jax.lax.top_k(logits, k)
    rw = jax.nn.softmax(top_logits, axis=-1)
    flat_e = indices.reshape(-1).astype(jnp.int32)
    flat_t = (jnp.arange(b * s * k, dtype=jnp.int32) // k)
    flat_w = rw.reshape(-1)
    order = jnp.argsort(flat_e)
    e_s = flat_e[order]
    t_s = flat_t[order]
    w_s = flat_w[order].reshape(-1, 1)

    npairs = b * s * k
    nut = u // _UT

    grid_spec = pltpu.PrefetchScalarGridSpec(
        num_scalar_prefetch=2,
        grid=(nut, npairs),
        in_specs=[
            pl.BlockSpec((b * s, d), lambda i, j, er, tr: (0, 0)),
            pl.BlockSpec((1, _UT, d), lambda i, j, er, tr: (er[j], i, 0)),
            pl.BlockSpec((1, d, _UT), lambda i, j, er, tr: (er[j], 0, i)),
            pl.BlockSpec((1, 1, _UT), lambda i, j, er, tr: (er[j], 0, i)),
            pl.BlockSpec((1, 1, d), lambda i, j, er, tr: (er[j], 0, 0)),
            pl.BlockSpec((npairs, 1), lambda i, j, er, tr: (0, 0)),
        ],
        out_specs=pl.BlockSpec((b * s, d), lambda i, j, er, tr: (0, 0)),
    )

    out = pl.pallas_call(
        _ffn_body,
        grid_spec=grid_spec,
        out_shape=jax.ShapeDtypeStruct((b * s, d), jnp.float32),
        compiler_params=pltpu.CompilerParams(
            dimension_semantics=("arbitrary", "arbitrary"),
        ),
    )(e_s, t_s, x2, W_up, W_down,
      b_up.reshape(e, 1, u), b_down.reshape(e, 1, d), w_s)
    return out.reshape(b, s, d)


# speedup vs baseline: 1.5696x; 1.1244x over previous
import functools

import jax
import jax.numpy as jnp
from jax.experimental import pallas as pl
from jax.experimental.pallas import tpu as pltpu

_K = 2
_R = 8
_GC = 0.7978845608028654


@functools.partial(jax.jit, static_argnames=())
def kernel(x, W_router, W_up, W_down, b_up, b_down):
    b, s, d = x.shape
    e, u, _ = W_up.shape
    k = _K
    bs = b * s
    npairs = bs * k
    x2 = x.reshape(bs, d)

    logits = x2 @ W_router
    top_logits, indices = jax.lax.top_k(logits, k)
    rw = jax.nn.softmax(top_logits, axis=-1)
    flat_e = indices.reshape(-1).astype(jnp.int32)
    flat_t = (jnp.arange(npairs, dtype=jnp.int32) // k)
    flat_w = rw.reshape(-1)
    order = jnp.argsort(flat_e)
    e_s = flat_e[order]
    t_s = flat_t[order]
    w_s = flat_w[order]

    jj = jnp.arange(npairs, dtype=jnp.int32)
    first_idx = jnp.searchsorted(e_s, e_s, side="left").astype(jnp.int32)
    r = jj - first_idx
    chunk = r // _R
    row = r % _R
    prev_e = jnp.concatenate([e_s[:1] - 1, e_s[:-1]])
    prev_c = jnp.concatenate([jnp.zeros((1,), jnp.int32) - 1, chunk[:-1]])
    slot_start = (e_s != prev_e) | (chunk != prev_c)
    slot_id = jnp.cumsum(slot_start.astype(jnp.int32)) - 1
    nslots = slot_id[-1] + 1
    slot_expert = jnp.full((npairs,), e_s[-1], jnp.int32).at[slot_id].set(e_s)
    nre = nslots.reshape(1)
    tok_idx = jnp.zeros((npairs, _R), jnp.int32).at[slot_id, row].set(t_s)
    xe = x2[tok_idx.reshape(-1)].reshape(npairs, _R, d)
    comb = jnp.zeros((bs, npairs * _R), jnp.float32
                     ).at[t_s, slot_id * _R + row].set(w_s)

    grid_spec = pltpu.PrefetchScalarGridSpec(
        num_scalar_prefetch=2,
        grid=(npairs,),
        in_specs=[
            pl.BlockSpec((1, _R, d), lambda g, se, nr: (g, 0, 0)),
            pl.BlockSpec((1, u, d), lambda g, se, nr: (se[g], 0, 0)),
            pl.BlockSpec((1, d, u), lambda g, se, nr: (se[g], 0, 0)),
            pl.BlockSpec((1, 1, u), lambda g, se, nr: (se[g], 0, 0)),
            pl.BlockSpec((1, 1, d), lambda g, se, nr: (se[g], 0, 0)),
        ],
        out_specs=pl.BlockSpec((1, _R, d), lambda g, se, nr: (g, 0, 0)),
    )

    def ffn_body(se_ref, nr_ref, xe_ref, wu_ref, wd_ref, bu_ref, bd_ref,
                 out2_ref):
        g = pl.program_id(0)
        active = g < nr_ref[0]

        @pl.when(active)
        def _slot():
            h = jax.lax.dot_general(xe_ref[0], wu_ref[0],
                                    (((1,), (1,)), ((), ())),
                                    preferred_element_type=jnp.float32)
            h = h + bu_ref[0]
            inner = _GC * (h + 0.044715 * (h * h * h))
            h = h * (1.0 / (1.0 + jnp.exp(-2.0 * inner)))
            o = jax.lax.dot_general(h, wd_ref[0],
                                    (((1,), (1,)), ((), ())),
                                    preferred_element_type=jnp.float32)
            out2_ref[0] = o + bd_ref[0]

        @pl.when(jnp.logical_not(active))
        def _pad():
            out2_ref[...] = jnp.zeros_like(out2_ref)

    out2 = pl.pallas_call(
        ffn_body,
        grid_spec=grid_spec,
        out_shape=jax.ShapeDtypeStruct((npairs, _R, d), jnp.float32),
        compiler_params=pltpu.CompilerParams(
            dimension_semantics=("arbitrary",),
        ),
    )(slot_expert, nre, xe, W_up, W_down,
      b_up.reshape(e, 1, u), b_down.reshape(e, 1, d))
    out = comb @ out2.reshape(npairs * _R, d)
    return out.reshape(b, s, d)
